# baseline (speedup 1.0000x reference)
import jax
import jax.numpy as jnp
from jax import lax
from jax.experimental import pallas as pl
from jax.experimental.pallas import tpu as pltpu
from jax.experimental.pallas import tpu_sc as plsc

_DIM = 128
_EPS = 1e-12

_NC, _NS = 2, 16
_NW = _NC * _NS
_N = 4 * 8192
_PER_W = _N // _NW
_CHUNK = 64
_NCHUNK = _PER_W // _CHUNK
_NRB = 6
_NOB = 4

_L = 16
_J = _DIM // _L


def _sc_kernel(ids_hbm, ttid_hbm, word_hbm, type_hbm, out_hbm,
               idx_v, ttid_v, type_v, rbufs, obufs, gsems, osems):
    wid = lax.axis_index("s") * _NC + lax.axis_index("c")
    base = wid * _PER_W

    def fire_gather(c):
        return pltpu.async_copy(
            word_hbm.at[idx_v.at[c]], rbufs[c % _NRB], gsems[c % _NRB])

    pltpu.sync_copy(ids_hbm.at[pl.ds(wid * _NCHUNK, _NCHUNK)], idx_v)
    gcps = {c: fire_gather(c) for c in range(min(_NRB, _NCHUNK))}
    ocps = {}

    pltpu.sync_copy(ttid_hbm.at[pl.ds(base, _PER_W)], ttid_v)
    pltpu.sync_copy(type_hbm, type_v)

    e0 = [type_v[0, pl.ds(_L * j, _L)] for j in range(_J)]
    dt = [type_v[1, pl.ds(_L * j, _L)] - e0[j] for j in range(_J)]

    for c in range(_NCHUNK):
        rb = rbufs[c % _NRB]
        ob = obufs[c % _NOB]
        gcps[c].wait()
        if c >= _NOB:
            ocps[c - _NOB].wait()

        @plsc.parallel_loop(0, _CHUNK, step=1, unroll=1)
        def _row(r):
            tid = plsc.load_gather(
                ttid_v,
                [jnp.full((_L,), c * _CHUNK + r, jnp.int32)]).astype(jnp.float32)
            xs = []
            acc = jnp.zeros((_L,), jnp.float32)
            for j in range(_J):
                x = rb[r, pl.ds(_L * j, _L)] + e0[j] + tid * dt[j]
                xs.append(x)
                acc = acc + x * x
            s = jnp.sum(acc)
            v = jnp.full((_L,), s, jnp.float32) * (1.0 / _DIM) + _EPS
            i = plsc.bitcast(v, jnp.int32)
            i = jnp.int32(0x5F3759DF) - lax.shift_right_logical(
                i, jnp.full((_L,), 1, jnp.int32))
            y = plsc.bitcast(i, jnp.float32)
            for _ in range(2):
                y = y * (1.5 - 0.5 * v * y * y)
            for j in range(_J):
                ob[r, pl.ds(_L * j, _L)] = xs[j] * y

        ocps[c] = pltpu.async_copy(
            ob, out_hbm.at[pl.ds(base + c * _CHUNK, _CHUNK)], osems[c % _NOB])
        if c + _NRB < _NCHUNK:
            gcps[c + _NRB] = fire_gather(c + _NRB)

    for c in range(max(0, _NCHUNK - _NOB), _NCHUNK):
        ocps[c].wait()


@jax.jit
def kernel(input_ids, token_type_ids, word_emb, type_emb):
    B, S = input_ids.shape
    ids2d = input_ids.reshape(_N // _CHUNK, _CHUNK)
    ttid = token_type_ids.reshape(_N)

    mesh = plsc.VectorSubcoreMesh(
        core_axis_name="c", subcore_axis_name="s",
        num_cores=_NC, num_subcores=_NS)
    out = pl.kernel(
        _sc_kernel,
        out_type=jax.ShapeDtypeStruct((_N, _DIM), jnp.float32),
        mesh=mesh,
        compiler_params=pltpu.CompilerParams(needs_layout_passes=False),
        scratch_types=[
            pltpu.VMEM((_NCHUNK, _CHUNK), jnp.int32),
            pltpu.VMEM((_PER_W,), jnp.int32),
            pltpu.VMEM((2, _DIM), jnp.float32),
            [pltpu.VMEM((_CHUNK, _DIM), jnp.float32) for _ in range(_NRB)],
            [pltpu.VMEM((_CHUNK, _DIM), jnp.float32) for _ in range(_NOB)],
            [pltpu.SemaphoreType.DMA for _ in range(_NRB)],
            [pltpu.SemaphoreType.DMA for _ in range(_NOB)],
        ],
    )(ids2d, ttid, word_emb, type_emb)
    return out.reshape(B, S, _DIM)


# speedup vs baseline: 1.0713x; 1.0713x over previous
import jax
import jax.numpy as jnp
from jax import lax
from jax.experimental import pallas as pl
from jax.experimental.pallas import tpu as pltpu
from jax.experimental.pallas import tpu_sc as plsc

_DIM = 128
_EPS = 1e-12

_NC, _NS = 2, 16
_NW = _NC * _NS
_N = 4 * 8192
_PER_W = _N // _NW
_CHUNK = 128
_NCHUNK = _PER_W // _CHUNK
_NRB = 4
_NOB = 2

_L = 16
_J = _DIM // _L


def _sc_kernel(ids_hbm, ttid_hbm, word_hbm, type_hbm, out_hbm,
               idx_v, ttid_v, type_v, rbufs, obufs, gsems, osems):
    wid = lax.axis_index("s") * _NC + lax.axis_index("c")
    base = wid * _PER_W

    def fire_gather(c):
        return pltpu.async_copy(
            word_hbm.at[idx_v.at[c]], rbufs[c % _NRB], gsems[c % _NRB])

    pltpu.sync_copy(ids_hbm.at[pl.ds(wid * _NCHUNK, _NCHUNK)], idx_v)
    gcps = {c: fire_gather(c) for c in range(min(_NRB, _NCHUNK))}
    ocps = {}

    pltpu.sync_copy(ttid_hbm.at[pl.ds(base, _PER_W)], ttid_v)
    pltpu.sync_copy(type_hbm, type_v)

    e0 = [type_v[0, pl.ds(_L * j, _L)] for j in range(_J)]
    dt = [type_v[1, pl.ds(_L * j, _L)] - e0[j] for j in range(_J)]

    for c in range(_NCHUNK):
        rb = rbufs[c % _NRB]
        ob = obufs[c % _NOB]
        gcps[c].wait()
        if c >= _NOB:
            ocps[c - _NOB].wait()

        @plsc.parallel_loop(0, _CHUNK, step=1, unroll=1)
        def _row(r):
            tid = plsc.load_gather(
                ttid_v,
                [jnp.full((_L,), c * _CHUNK + r, jnp.int32)]).astype(jnp.float32)
            xs = []
            acc = jnp.zeros((_L,), jnp.float32)
            for j in range(_J):
                x = rb[r, pl.ds(_L * j, _L)] + e0[j] + tid * dt[j]
                xs.append(x)
                acc = acc + x * x
            s = jnp.sum(acc) * (1.0 / _DIM) + _EPS
            i = lax.bitcast_convert_type(s, jnp.int32)
            i = jnp.int32(0x5F3759DF) - lax.shift_right_logical(i, 1)
            y = lax.bitcast_convert_type(i, jnp.float32)
            for _ in range(2):
                y = y * (1.5 - 0.5 * s * y * y)
            ybc = jnp.full((_L,), y, jnp.float32)
            for j in range(_J):
                ob[r, pl.ds(_L * j, _L)] = xs[j] * ybc

        ocps[c] = pltpu.async_copy(
            ob, out_hbm.at[pl.ds(base + c * _CHUNK, _CHUNK)], osems[c % _NOB])
        if c + _NRB < _NCHUNK:
            gcps[c + _NRB] = fire_gather(c + _NRB)

    for c in range(max(0, _NCHUNK - _NOB), _NCHUNK):
        ocps[c].wait()


@jax.jit
def kernel(input_ids, token_type_ids, word_emb, type_emb):
    B, S = input_ids.shape
    ids2d = input_ids.reshape(_N // _CHUNK, _CHUNK)
    ttid = token_type_ids.reshape(_N)

    mesh = plsc.VectorSubcoreMesh(
        core_axis_name="c", subcore_axis_name="s",
        num_cores=_NC, num_subcores=_NS)
    out = pl.kernel(
        _sc_kernel,
        out_type=jax.ShapeDtypeStruct((_N, _DIM), jnp.float32),
        mesh=mesh,
        compiler_params=pltpu.CompilerParams(needs_layout_passes=False),
        scratch_types=[
            pltpu.VMEM((_NCHUNK, _CHUNK), jnp.int32),
            pltpu.VMEM((_PER_W,), jnp.int32),
            pltpu.VMEM((2, _DIM), jnp.float32),
            [pltpu.VMEM((_CHUNK, _DIM), jnp.float32) for _ in range(_NRB)],
            [pltpu.VMEM((_CHUNK, _DIM), jnp.float32) for _ in range(_NOB)],
            [pltpu.SemaphoreType.DMA for _ in range(_NRB)],
            [pltpu.SemaphoreType.DMA for _ in range(_NOB)],
        ],
    )(ids2d, ttid, word_emb, type_emb)
    return out.reshape(B, S, _DIM)


# speedup vs baseline: 1.1200x; 1.0455x over previous
import jax
import jax.numpy as jnp
from jax import lax
from jax.experimental import pallas as pl
from jax.experimental.pallas import tpu as pltpu
from jax.experimental.pallas import tpu_sc as plsc

_DIM = 128
_EPS = 1e-12

_NC, _NS = 2, 16
_NW = _NC * _NS
_N = 4 * 8192
_PER_W = _N // _NW
_CHUNK = 128
_NCHUNK = _PER_W // _CHUNK
_NRB = 4
_NOB = 2

_L = 16
_J = _DIM // _L


def _sc_kernel(ids_hbm, ttid_hbm, word_hbm, type_hbm, out_hbm,
               idx_v, ttid_v, type_v, rbufs, obufs, gsems, osems):
    wid = lax.axis_index("s") * _NC + lax.axis_index("c")
    base = wid * _PER_W

    def fire_gather(c):
        return pltpu.async_copy(
            word_hbm.at[idx_v.at[c]], rbufs[c % _NRB], gsems[c % _NRB])

    pltpu.sync_copy(ids_hbm.at[pl.ds(wid * _NCHUNK, _NCHUNK)], idx_v)
    gcps = {c: fire_gather(c) for c in range(min(_NRB, _NCHUNK))}
    ocps = {}

    pltpu.sync_copy(ttid_hbm.at[pl.ds(base, _PER_W)], ttid_v)
    pltpu.sync_copy(type_hbm, type_v)

    e0 = [type_v[0, pl.ds(_L * j, _L)] for j in range(_J)]
    dt = [type_v[1, pl.ds(_L * j, _L)] - e0[j] for j in range(_J)]

    for c in range(_NCHUNK):
        rb = rbufs[c % _NRB]
        ob = obufs[c % _NOB]
        gcps[c].wait()
        if c >= _NOB:
            ocps[c - _NOB].wait()

        @plsc.parallel_loop(0, _CHUNK, step=1, unroll=1)
        def _row(r):
            tid = plsc.load_gather(
                ttid_v,
                [jnp.full((_L,), c * _CHUNK + r, jnp.int32)]).astype(jnp.float32)
            xs = []
            acc = jnp.zeros((_L,), jnp.float32)
            for j in range(_J):
                x = rb[r, pl.ds(_L * j, _L)] + e0[j] + tid * dt[j]
                xs.append(x)
                acc = acc + x * x
            s = jnp.sum(acc) * (1.0 / _DIM) + _EPS
            i = lax.bitcast_convert_type(s, jnp.int32)
            i = jnp.int32(0x5F3759DF) - lax.shift_right_logical(i, 1)
            y = lax.bitcast_convert_type(i, jnp.float32)
            for _ in range(1):
                y = y * (1.5 - 0.5 * s * y * y)
            ybc = jnp.full((_L,), y, jnp.float32)
            for j in range(_J):
                ob[r, pl.ds(_L * j, _L)] = xs[j] * ybc

        ocps[c] = pltpu.async_copy(
            ob, out_hbm.at[pl.ds(base + c * _CHUNK, _CHUNK)], osems[c % _NOB])
        if c + _NRB < _NCHUNK:
            gcps[c + _NRB] = fire_gather(c + _NRB)

    for c in range(max(0, _NCHUNK - _NOB), _NCHUNK):
        ocps[c].wait()


@jax.jit
def kernel(input_ids, token_type_ids, word_emb, type_emb):
    B, S = input_ids.shape
    ids2d = input_ids.reshape(_N // _CHUNK, _CHUNK)
    ttid = token_type_ids.reshape(_N)

    mesh = plsc.VectorSubcoreMesh(
        core_axis_name="c", subcore_axis_name="s",
        num_cores=_NC, num_subcores=_NS)
    out = pl.kernel(
        _sc_kernel,
        out_type=jax.ShapeDtypeStruct((_N, _DIM), jnp.float32),
        mesh=mesh,
        compiler_params=pltpu.CompilerParams(needs_layout_passes=False),
        scratch_types=[
            pltpu.VMEM((_NCHUNK, _CHUNK), jnp.int32),
            pltpu.VMEM((_PER_W,), jnp.int32),
            pltpu.VMEM((2, _DIM), jnp.float32),
            [pltpu.VMEM((_CHUNK, _DIM), jnp.float32) for _ in range(_NRB)],
            [pltpu.VMEM((_CHUNK, _DIM), jnp.float32) for _ in range(_NOB)],
            [pltpu.SemaphoreType.DMA for _ in range(_NRB)],
            [pltpu.SemaphoreType.DMA for _ in range(_NOB)],
        ],
    )(ids2d, ttid, word_emb, type_emb)
    return out.reshape(B, S, _DIM)
